# baseline (speedup 1.0000x reference)
import functools

import jax
import jax.numpy as jnp
from jax import lax
from jax.experimental import pallas as pl
from jax.experimental.pallas import tpu as pltpu
from jax.experimental.pallas import tpu_sc as plsc

_D_MODEL = 2048
_N_EXPERTS = 64
_BLK = 2048
_TOKENS = 16384
_NC = 2
_NS = 16
_TPW = _TOKENS // (_NC * _NS)
_GRP = _TPW // 16


def _dense_body(x_ref, w_ref, probs_ref, logits_t_ref):
    x = x_ref[...]
    w = w_ref[...]
    logits = jax.lax.dot_general(
        x, w, (((1,), (1,)), ((), ())), preferred_element_type=jnp.float32
    )
    logits_t_ref[...] = logits.T
    m1 = jnp.max(logits, axis=-1, keepdims=True)
    p = jnp.exp(logits - m1)
    probs_ref[...] = p / jnp.sum(p, axis=-1, keepdims=True)


def _interleave(a, b):
    lane = lax.iota(jnp.int32, 16)
    half = lax.shift_right_logical(lane, 1)
    even = (lane & 1) == 0
    dnums = lax.GatherDimensionNumbers(
        offset_dims=(), collapsed_slice_dims=(0,), start_index_map=(0,)
    )

    def permute(v, ids):
        return lax.gather(
            v, ids[:, None], dnums, (1,),
            mode=lax.GatherScatterMode.PROMISE_IN_BOUNDS,
        )

    lo = jnp.where(even, permute(a, half), permute(b, half))
    hi = jnp.where(even, permute(a, half + 8), permute(b, half + 8))
    return lo, hi


def _sc_body(logits_t_hbm, idx_hbm, scores_hbm, lg_v, idx_v, sc_v):
    wid = lax.axis_index("s") * _NC + lax.axis_index("c")
    t0 = wid * _TPW
    pltpu.sync_copy(logits_t_hbm.at[:, pl.ds(t0, _TPW)], lg_v)

    def group(g, carry):
        o = g * 16
        m1 = lg_v[0, pl.ds(o, 16)]
        i1 = jnp.zeros((16,), jnp.int32)
        m2 = jnp.full((16,), -3.0e38, jnp.float32)
        i2 = jnp.zeros((16,), jnp.int32)
        for e in range(1, _N_EXPERTS):
            v = lg_v[e, pl.ds(o, 16)]
            ev = jnp.full((16,), e, jnp.int32)
            gt1 = v > m1
            gt2 = v > m2
            i2 = jnp.where(gt1, i1, jnp.where(gt2, ev, i2))
            m2 = jnp.where(gt1, m1, jnp.where(gt2, v, m2))
            i1 = jnp.where(gt1, ev, i1)
            m1 = jnp.where(gt1, v, m1)
        e2 = jnp.exp(m2 - m1)
        den = 1.0 + e2
        ilo, ihi = _interleave(i1, i2)
        idx_v[pl.ds(2 * o, 16)] = ilo
        idx_v[pl.ds(2 * o + 16, 16)] = ihi
        slo, shi = _interleave(1.0 / den, e2 / den)
        sc_v[pl.ds(2 * o, 16)] = slo
        sc_v[pl.ds(2 * o + 16, 16)] = shi
        return carry

    lax.fori_loop(0, _GRP, group, 0)
    pltpu.sync_copy(idx_v, idx_hbm.at[pl.ds(2 * t0, 2 * _TPW)])
    pltpu.sync_copy(sc_v, scores_hbm.at[pl.ds(2 * t0, 2 * _TPW)])


_sc_top2 = pl.kernel(
    _sc_body,
    out_type=[
        jax.ShapeDtypeStruct((2 * _TOKENS,), jnp.int32),
        jax.ShapeDtypeStruct((2 * _TOKENS,), jnp.float32),
    ],
    mesh=plsc.VectorSubcoreMesh(core_axis_name="c", subcore_axis_name="s"),
    scratch_types=[
        pltpu.VMEM((_N_EXPERTS, _TPW), jnp.float32),
        pltpu.VMEM((2 * _TPW,), jnp.int32),
        pltpu.VMEM((2 * _TPW,), jnp.float32),
    ],
)


@functools.partial(jax.jit, static_argnames=())
def kernel(x, W_gate):
    b, s, d = x.shape
    tokens = b * s
    x2 = x.reshape(tokens, d)
    grid = (tokens // _BLK,)
    probs, logits_t = pl.pallas_call(
        _dense_body,
        grid=grid,
        in_specs=[
            pl.BlockSpec((_BLK, d), lambda i: (i, 0)),
            pl.BlockSpec((_N_EXPERTS, d), lambda i: (0, 0)),
        ],
        out_specs=[
            pl.BlockSpec((_BLK, _N_EXPERTS), lambda i: (i, 0)),
            pl.BlockSpec((_N_EXPERTS, _BLK), lambda i: (0, i)),
        ],
        out_shape=[
            jax.ShapeDtypeStruct((tokens, _N_EXPERTS), jnp.float32),
            jax.ShapeDtypeStruct((_N_EXPERTS, tokens), jnp.float32),
        ],
    )(x2, W_gate)
    idx_flat, scores_flat = _sc_top2(logits_t)
    return (
        idx_flat.reshape(b, s, 2),
        scores_flat.reshape(b, s, 2),
        probs.reshape(b, s, _N_EXPERTS),
    )


# speedup vs baseline: 1.4411x; 1.4411x over previous
import functools

import jax
import jax.numpy as jnp
from jax import lax
from jax.experimental import pallas as pl
from jax.experimental.pallas import tpu as pltpu
from jax.experimental.pallas import tpu_sc as plsc

_D_MODEL = 2048
_N_EXPERTS = 64
_BLK = 2048
_TOKENS = 16384
_NC = 2
_NS = 16
_TPW = _TOKENS // (_NC * _NS)
_GRP = _TPW // 16


def _dense_body(x_ref, w_ref, probs_ref, logits_t_ref):
    x = x_ref[...]
    w = w_ref[...]
    logits = jax.lax.dot_general(
        x, w, (((1,), (1,)), ((), ())), preferred_element_type=jnp.float32
    )
    logits_t_ref[...] = logits.T
    m1 = jnp.max(logits, axis=-1, keepdims=True)
    p = jnp.exp(logits - m1)
    probs_ref[...] = p / jnp.sum(p, axis=-1, keepdims=True)


def _sc_body(logits_t_hbm, idx_hbm, scores_hbm, lg_v, idx_v, sc_v):
    wid = lax.axis_index("s") * _NC + lax.axis_index("c")
    t0 = wid * _TPW
    pltpu.sync_copy(logits_t_hbm.at[:, pl.ds(t0, _TPW)], lg_v)

    def group(g, carry):
        o = g * 16
        m1 = lg_v[0, pl.ds(o, 16)]
        i1 = jnp.zeros((16,), jnp.int32)
        m2 = jnp.full((16,), -3.0e38, jnp.float32)
        i2 = jnp.zeros((16,), jnp.int32)
        for e in range(1, _N_EXPERTS):
            v = lg_v[e, pl.ds(o, 16)]
            ev = jnp.full((16,), e, jnp.int32)
            gt1 = v > m1
            gt2 = v > m2
            i2 = jnp.where(gt1, i1, jnp.where(gt2, ev, i2))
            m2 = jnp.where(gt1, m1, jnp.where(gt2, v, m2))
            i1 = jnp.where(gt1, ev, i1)
            m1 = jnp.where(gt1, v, m1)
        e2 = jnp.exp(m2 - m1)
        den = 1.0 + e2
        idx_v[0, pl.ds(o, 16)] = i1
        idx_v[1, pl.ds(o, 16)] = i2
        sc_v[0, pl.ds(o, 16)] = 1.0 / den
        sc_v[1, pl.ds(o, 16)] = e2 / den
        return carry

    lax.fori_loop(0, _GRP, group, 0)
    pltpu.sync_copy(idx_v, idx_hbm.at[:, pl.ds(t0, _TPW)])
    pltpu.sync_copy(sc_v, scores_hbm.at[:, pl.ds(t0, _TPW)])


_sc_top2 = pl.kernel(
    _sc_body,
    out_type=[
        jax.ShapeDtypeStruct((2, _TOKENS), jnp.int32),
        jax.ShapeDtypeStruct((2, _TOKENS), jnp.float32),
    ],
    mesh=plsc.VectorSubcoreMesh(core_axis_name="c", subcore_axis_name="s"),
    scratch_types=[
        pltpu.VMEM((_N_EXPERTS, _TPW), jnp.float32),
        pltpu.VMEM((2, _TPW), jnp.int32),
        pltpu.VMEM((2, _TPW), jnp.float32),
    ],
)


@functools.partial(jax.jit, static_argnames=())
def kernel(x, W_gate):
    b, s, d = x.shape
    tokens = b * s
    x2 = x.reshape(tokens, d)
    grid = (tokens // _BLK,)
    probs, logits_t = pl.pallas_call(
        _dense_body,
        grid=grid,
        in_specs=[
            pl.BlockSpec((_BLK, d), lambda i: (i, 0)),
            pl.BlockSpec((_N_EXPERTS, d), lambda i: (0, 0)),
        ],
        out_specs=[
            pl.BlockSpec((_BLK, _N_EXPERTS), lambda i: (i, 0)),
            pl.BlockSpec((_N_EXPERTS, _BLK), lambda i: (0, i)),
        ],
        out_shape=[
            jax.ShapeDtypeStruct((tokens, _N_EXPERTS), jnp.float32),
            jax.ShapeDtypeStruct((_N_EXPERTS, tokens), jnp.float32),
        ],
    )(x2, W_gate)
    idx_p, scores_p = _sc_top2(logits_t)
    return (
        jnp.swapaxes(idx_p, 0, 1).reshape(b, s, 2),
        jnp.swapaxes(scores_p, 0, 1).reshape(b, s, 2),
        probs.reshape(b, s, _N_EXPERTS),
    )
